# baseline (speedup 1.0000x reference)
import functools

import jax
import jax.numpy as jnp
from jax import lax
from jax.experimental import pallas as pl
from jax.experimental.pallas import tpu as pltpu
from jax.experimental.pallas import tpu_sc as plsc

N_NODES = 100000
N_CLAUSES = 100000
E = 3200000

NC = 2
NS = 16
L = 16
NW = NC * NS
CH = 2560
NCHUNK = E // CH
FULL = NCHUNK // NW
REM = NCHUNK - FULL * NW
GRP = CH // L
PKW = 3136
NCP = 100352


def _pack_body(xvt_ref, out_ref):
    x = xvt_ref[...]
    shifts = lax.broadcasted_iota(jnp.int32, (32, PKW), 0)
    bits = jnp.where(x >= 0.5, jnp.left_shift(jnp.int32(1), shifts),
                     jnp.int32(0))
    out_ref[...] = jnp.sum(bits, axis=0, keepdims=True)


def _reduce_body(h_ref, o_ref):
    s = h_ref[pl.ds(0, NCP)]
    for r in range(1, NW):
        s = s + h_ref[pl.ds(r * NCP, NCP)]
    o_ref[0, 0] = jnp.min(s[:N_CLAUSES]).astype(jnp.float32)


def _hist_body(pos_hbm, neg_hbm, pk_hbm, out_hbm,
               hist, pk, buf0, buf1, sem0, sem1):
    wid = lax.axis_index("s") * NC + lax.axis_index("c")
    nk = FULL + jnp.where(wid < REM, 1, 0)

    def start(adj_hbm, j, buf, sem):
        off = (wid + j * NW) * CH
        pltpu.make_async_copy(adj_hbm.at[:, pl.ds(off, CH)], buf,
                              sem).start()

    def drain(adj_hbm, buf, sem):
        pltpu.make_async_copy(adj_hbm.at[:, pl.ds(0, CH)], buf,
                              sem).wait()

    start(pos_hbm, 0, buf0, sem0)
    start(pos_hbm, 1, buf1, sem1)

    pltpu.sync_copy(pk_hbm.at[0], pk)

    zero = jnp.zeros((L,), jnp.int32)

    @plsc.parallel_loop(0, NCP // L, unroll=16)
    def _(i):
        hist[pl.ds(i * L, L)] = zero

    ones = jnp.ones((L,), jnp.int32)

    def run_groups(buf, want_bit):
        @plsc.parallel_loop(0, GRP, unroll=32)
        def _(g):
            nid = buf[1, pl.ds(g * L, L)]
            cid = buf[0, pl.ds(g * L, L)]
            w = plsc.load_gather(pk, [lax.shift_right_logical(nid, 5)])
            bit = jnp.bitwise_and(
                lax.shift_right_logical(w, jnp.bitwise_and(nid, 31)), 1)
            plsc.addupdate_scatter(hist, [cid], ones, mask=bit == want_bit)

    def process(adj_hbm, want_bit, primed):
        if not primed:
            start(adj_hbm, 0, buf0, sem0)
            start(adj_hbm, 1, buf1, sem1)

        def pair_body(j, _):
            j0 = 2 * j

            @pl.when(j0 < nk)
            def _():
                drain(adj_hbm, buf0, sem0)
                run_groups(buf0, want_bit)

            @pl.when(j0 + 2 < nk)
            def _():
                start(adj_hbm, j0 + 2, buf0, sem0)

            @pl.when(j0 + 1 < nk)
            def _():
                drain(adj_hbm, buf1, sem1)
                run_groups(buf1, want_bit)

            @pl.when(j0 + 3 < nk)
            def _():
                start(adj_hbm, j0 + 3, buf1, sem1)

            return 0

        lax.fori_loop(0, (FULL + 2) // 2, pair_body, 0)

    process(pos_hbm, 1, True)
    process(neg_hbm, 0, False)

    pltpu.sync_copy(hist, out_hbm.at[pl.ds(wid * NCP, NCP)])


_hist_kernel = functools.partial(
    pl.kernel,
    out_type=jax.ShapeDtypeStruct((NW * NCP,), jnp.int32),
    mesh=plsc.VectorSubcoreMesh(core_axis_name="c", subcore_axis_name="s"),
    compiler_params=pltpu.CompilerParams(needs_layout_passes=False,
                                         use_tc_tiling_on_sc=True),
    scratch_types=[
        pltpu.VMEM((NCP,), jnp.int32),
        pltpu.VMEM((PKW,), jnp.int32),
        pltpu.VMEM((2, CH), jnp.int32),
        pltpu.VMEM((2, CH), jnp.int32),
        pltpu.SemaphoreType.DMA,
        pltpu.SemaphoreType.DMA,
    ],
)(_hist_body)


def kernel(xv, adj_pos, adj_neg):
    xvp = jnp.concatenate(
        [xv, jnp.zeros((PKW * 32 - N_NODES,), jnp.float32)])
    xvt = xvp.reshape(PKW, 32).T
    pk = pl.pallas_call(
        _pack_body,
        out_shape=jax.ShapeDtypeStruct((1, PKW), jnp.int32),
    )(xvt)
    hist = _hist_kernel(adj_pos, adj_neg, pk)
    out = pl.pallas_call(
        _reduce_body,
        out_shape=jax.ShapeDtypeStruct((1, 1), jnp.float32),
        out_specs=pl.BlockSpec(memory_space=pltpu.SMEM),
    )(hist)
    return out[0, 0]


# speedup vs baseline: 1.0019x; 1.0019x over previous
import functools

import jax
import jax.numpy as jnp
from jax import lax
from jax.experimental import pallas as pl
from jax.experimental.pallas import tpu as pltpu
from jax.experimental.pallas import tpu_sc as plsc

N_NODES = 100000
N_CLAUSES = 100000
E = 3200000

NC = 2
NS = 16
L = 16
NW = NC * NS
CH = 2560
NCHUNK = E // CH
FULL = NCHUNK // NW
REM = NCHUNK - FULL * NW
GRP = CH // L
PKW = 3136
NCP = 100352


def _pack_body(xvt_ref, out_ref):
    x = xvt_ref[...]
    shifts = lax.broadcasted_iota(jnp.int32, (32, PKW), 0)
    bits = jnp.where(x >= 0.5, jnp.left_shift(jnp.int32(1), shifts),
                     jnp.int32(0))
    out_ref[...] = jnp.sum(bits, axis=0, keepdims=True)


def _reduce_body(h_ref, o_ref):
    s = h_ref[pl.ds(0, NCP)]
    for r in range(1, NW):
        s = s + h_ref[pl.ds(r * NCP, NCP)]
    o_ref[0, 0] = jnp.min(s[:N_CLAUSES]).astype(jnp.float32)


def _hist_body(pos_hbm, neg_hbm, pk_hbm, out_hbm,
               hist, pk, buf0, buf1, sem0, sem1):
    wid = lax.axis_index("s") * NC + lax.axis_index("c")
    nk = FULL + jnp.where(wid < REM, 1, 0)

    def start(adj_hbm, j, buf, sem):
        off = (wid + j * NW) * CH
        pltpu.make_async_copy(adj_hbm.at[:, pl.ds(off, CH)], buf,
                              sem).start()

    def drain(adj_hbm, buf, sem):
        pltpu.make_async_copy(adj_hbm.at[:, pl.ds(0, CH)], buf,
                              sem).wait()

    start(pos_hbm, 0, buf0, sem0)
    start(pos_hbm, 1, buf1, sem1)

    pltpu.sync_copy(pk_hbm.at[0], pk)

    zero = jnp.zeros((L,), jnp.int32)

    @plsc.parallel_loop(0, NCP // L, unroll=16)
    def _(i):
        hist[pl.ds(i * L, L)] = zero

    ones = jnp.ones((L,), jnp.int32)

    def run_groups(buf, want_bit):
        @plsc.parallel_loop(0, GRP, unroll=16)
        def _(g):
            nid = buf[1, pl.ds(g * L, L)]
            cid = buf[0, pl.ds(g * L, L)]
            w = plsc.load_gather(pk, [lax.shift_right_logical(nid, 5)])
            bit = jnp.bitwise_and(
                lax.shift_right_logical(w, jnp.bitwise_and(nid, 31)), 1)
            plsc.addupdate_scatter(hist, [cid], ones, mask=bit == want_bit)

    def process(adj_hbm, want_bit, primed):
        if not primed:
            start(adj_hbm, 0, buf0, sem0)
            start(adj_hbm, 1, buf1, sem1)

        def pair_body(j, _):
            j0 = 2 * j

            @pl.when(j0 < nk)
            def _():
                drain(adj_hbm, buf0, sem0)
                run_groups(buf0, want_bit)

            @pl.when(j0 + 2 < nk)
            def _():
                start(adj_hbm, j0 + 2, buf0, sem0)

            @pl.when(j0 + 1 < nk)
            def _():
                drain(adj_hbm, buf1, sem1)
                run_groups(buf1, want_bit)

            @pl.when(j0 + 3 < nk)
            def _():
                start(adj_hbm, j0 + 3, buf1, sem1)

            return 0

        lax.fori_loop(0, (FULL + 2) // 2, pair_body, 0)

    process(pos_hbm, 1, True)
    process(neg_hbm, 0, False)

    pltpu.sync_copy(hist, out_hbm.at[pl.ds(wid * NCP, NCP)])


_hist_kernel = functools.partial(
    pl.kernel,
    out_type=jax.ShapeDtypeStruct((NW * NCP,), jnp.int32),
    mesh=plsc.VectorSubcoreMesh(core_axis_name="c", subcore_axis_name="s"),
    compiler_params=pltpu.CompilerParams(needs_layout_passes=False,
                                         use_tc_tiling_on_sc=True),
    scratch_types=[
        pltpu.VMEM((NCP,), jnp.int32),
        pltpu.VMEM((PKW,), jnp.int32),
        pltpu.VMEM((2, CH), jnp.int32),
        pltpu.VMEM((2, CH), jnp.int32),
        pltpu.SemaphoreType.DMA,
        pltpu.SemaphoreType.DMA,
    ],
)(_hist_body)


def kernel(xv, adj_pos, adj_neg):
    xvp = jnp.concatenate(
        [xv, jnp.zeros((PKW * 32 - N_NODES,), jnp.float32)])
    xvt = xvp.reshape(PKW, 32).T
    pk = pl.pallas_call(
        _pack_body,
        out_shape=jax.ShapeDtypeStruct((1, PKW), jnp.int32),
    )(xvt)
    hist = _hist_kernel(adj_pos, adj_neg, pk)
    out = pl.pallas_call(
        _reduce_body,
        out_shape=jax.ShapeDtypeStruct((1, 1), jnp.float32),
        out_specs=pl.BlockSpec(memory_space=pltpu.SMEM),
    )(hist)
    return out[0, 0]
